# baseline (speedup 1.0000x reference)
import jax
import jax.numpy as jnp
from jax import lax
from jax.experimental import pallas as pl
from jax.experimental.pallas import tpu as pltpu
from jax.experimental.pallas import tpu_sc as plsc

B = 16
L = 2048
D = 300
PR = D * B
DT = 240
RT0 = DT * B

RPB = 384
NI = RT0 // (2 * RPB)

NC = 2
NS = 16
LANES = 16
NW = NC * NS
RPW = (PR - RT0) // NW
RPAD = ((RPW + LANES - 1) // LANES) * LANES
RB = 8
BUCKET = 512
STRIPE = 8


def _tc_body(seq_a, seq_b, lenbc_ref, out_a, out_b, mask_ref):
    @pl.when(pl.program_id(0) == 0)
    def _():
        ln = lenbc_ref[:, 0:1]
        pos = lax.broadcasted_iota(jnp.int32, (RPB, L), 1).astype(jnp.float32)
        mask_ref[...] = jnp.where(pos < ln, 1.0, 0.0)

    ones = jnp.ones((L, 1), jnp.float32)
    m = mask_ref[...]
    for ref, oref in ((seq_a, out_a), (seq_b, out_b)):
        y = ref[...] * m
        res = jax.lax.dot_general(y, ones, (((1,), (0,)), ((), ())),
                                  preferred_element_type=jnp.float32)
        res = res / lenbc_ref[:, 0:1]
        oref[...] = jnp.broadcast_to(res, (RPB, 128)).reshape(1, RPB, 128)


def _mean_tc(seqT, len_bc):
    return pl.pallas_call(
        _tc_body,
        grid=(NI,),
        in_specs=[
            pl.BlockSpec((RPB, L), lambda i: (i, 0)),
            pl.BlockSpec((RPB, L), lambda i: (i + NI, 0)),
            pl.BlockSpec((RPB, 128), lambda i: (0, 0)),
        ],
        out_specs=[
            pl.BlockSpec((1, RPB, 128), lambda i: (i, 0, 0)),
            pl.BlockSpec((1, RPB, 128), lambda i: (i, 0, 0)),
        ],
        out_shape=[
            jax.ShapeDtypeStruct((NI, RPB, 128), jnp.float32),
            jax.ShapeDtypeStruct((NI, RPB, 128), jnp.float32),
        ],
        scratch_shapes=[pltpu.VMEM((RPB, L), jnp.float32)],
    )(seqT, seqT, len_bc)


def _sc_body(seq, len_hbm, out_hbm, bufs, vals, vals2, len_vm, lenf_vm,
             *sems):
    c = lax.axis_index("c")
    s = lax.axis_index("s")
    w = c * NS + s
    rbase = RT0 + w * RPW
    lane = lax.iota(jnp.int32, LANES)

    pltpu.sync_copy(len_hbm, len_vm.at[pl.ds(0, B)])
    pltpu.sync_copy(len_hbm, len_vm.at[pl.ds(B, B)])
    lenf_vm[pl.ds(0, LANES)] = len_vm[pl.ds(0, LANES)].astype(jnp.float32)
    lenf_vm[pl.ds(LANES, LANES)] = len_vm[pl.ds(LANES, LANES)].astype(
        jnp.float32)

    def row_of(k):
        return jnp.minimum(rbase + k, PR - 1)

    def len_of(k):
        return len_vm[pl.ds(row_of(k) & (B - 1), LANES)][0]

    def issue(k, b):
        r = row_of(k)
        n = len_of(k)
        for t in range(L // BUCKET):
            sz = (t + 1) * BUCKET

            @pl.when((n > t * BUCKET) & (n <= sz))
            def _():
                pltpu.async_copy(seq.at[r, pl.ds(0, sz)],
                                 bufs.at[b, pl.ds(0, sz)], sems[b])

    def drain(k, b):
        n = len_of(k)
        for t in range(L // BUCKET):
            sz = (t + 1) * BUCKET

            @pl.when((n > t * BUCKET) & (n <= sz))
            def _():
                pltpu.make_async_copy(seq.at[0, pl.ds(0, sz)],
                                      bufs.at[b, pl.ds(0, sz)],
                                      sems[b]).wait()

    zeros = jnp.zeros((LANES,), jnp.float32)

    def compute(k, b):
        n = len_of(k)

        def oct_body(q, a):
            base = q * (STRIPE * LANES)
            return tuple(a[j] + bufs[b, pl.ds(base + j * LANES, LANES)]
                         for j in range(STRIPE))

        a = lax.fori_loop(0, n // (STRIPE * LANES), oct_body,
                          (zeros,) * STRIPE)
        acc = ((a[0] + a[1]) + (a[2] + a[3])) + (
            (a[4] + a[5]) + (a[6] + a[7]))

        def single_body(j, acc):
            return acc + bufs[b, pl.ds(j * LANES, LANES)]

        nfull = n // LANES
        acc = lax.fori_loop((n // (STRIPE * LANES)) * STRIPE, nfull,
                            single_body, acc)
        xt = bufs[b, pl.ds(nfull * LANES, LANES)]
        acc = acc + jnp.where(lane < (n & (LANES - 1)), xt, 0.0)
        vals[pl.ds(k * LANES, LANES)] = plsc.cumsum(acc)

    for b in range(RB):
        issue(b, b)

    def octet(q, _):
        for b in range(RB):
            k = q * RB + b
            drain(k, b)
            compute(k, b)

            @pl.when(k + RB < RPAD)
            def _():
                issue(k + RB, b)
        return 0

    lax.fori_loop(0, RPAD // RB, octet, 0)

    for g in range(RPAD // LANES):
        idx = g * (LANES * LANES) + lane * LANES + (LANES - 1)
        tot = plsc.load_gather(vals, [idx])
        nvec = lenf_vm[pl.ds((rbase + g * LANES) & (B - 1), LANES)]
        vals2[pl.ds(g * LANES, LANES)] = tot / nvec
    pltpu.sync_copy(vals2, out_hbm.at[pl.ds(RPAD * w, RPAD)])


def _mean_sc(seqT, len32):
    mesh = plsc.VectorSubcoreMesh(
        core_axis_name="c", subcore_axis_name="s", num_cores=NC,
        num_subcores=NS)
    return pl.kernel(
        _sc_body,
        out_type=jax.ShapeDtypeStruct((NW * RPAD,), jnp.float32),
        mesh=mesh,
        compiler_params=pltpu.CompilerParams(use_tc_tiling_on_sc=True,
                                             needs_layout_passes=False),
        scratch_types=[
            pltpu.VMEM((RB, L), jnp.float32),
            pltpu.VMEM((RPAD * LANES,), jnp.float32),
            pltpu.VMEM((RPAD,), jnp.float32),
            pltpu.VMEM((2 * B,), jnp.int32),
            pltpu.VMEM((2 * B,), jnp.float32),
        ] + [pltpu.SemaphoreType.DMA] * RB,
    )(seqT, len32)


def kernel(sequences, lengths):
    seqT = sequences.transpose(2, 0, 1).reshape(PR, L)
    len32 = lengths.astype(jnp.int32)
    lenf = len32.astype(jnp.float32)
    len_bc = jnp.broadcast_to(
        jnp.tile(lenf, RPB // B)[:, None], (RPB, 128))

    sc = _mean_sc(seqT, len32)
    ta, tb = _mean_tc(seqT, len_bc)

    phys = jnp.concatenate([
        ta[:, :, 0].reshape(RT0 // 2),
        tb[:, :, 0].reshape(RT0 // 2),
        sc.reshape(NW, RPAD)[:, :RPW].reshape(PR - RT0),
    ])
    return phys.reshape(D, B).T


# speedup vs baseline: 1.8221x; 1.8221x over previous
import jax
import jax.numpy as jnp
from jax import lax
from jax.experimental import pallas as pl
from jax.experimental.pallas import tpu as pltpu

B = 16
L = 2048
D = 300
PR = D * B
RPB = 320
NI = PR // (3 * RPB)


def _tc_body(seq_a, seq_b, seq_c, lenbc_ref, out_a, out_b, out_c, mask_ref):
    @pl.when(pl.program_id(0) == 0)
    def _():
        ln = lenbc_ref[:, 0:1]
        pos = lax.broadcasted_iota(jnp.int32, (RPB, L), 1).astype(jnp.float32)
        mask_ref[...] = jnp.where(pos < ln, 1.0, 0.0)

    ones = jnp.ones((L, 1), jnp.float32)
    m = mask_ref[...]
    for ref, oref in ((seq_a, out_a), (seq_b, out_b), (seq_c, out_c)):
        y = ref[...] * m
        res = jax.lax.dot_general(y, ones, (((1,), (0,)), ((), ())),
                                  preferred_element_type=jnp.float32)
        res = res / lenbc_ref[:, 0:1]
        oref[...] = jnp.broadcast_to(res, (RPB, 128)).reshape(1, RPB, 128)


def _mean_tc(seqT, len_bc):
    return pl.pallas_call(
        _tc_body,
        grid=(NI,),
        in_specs=[
            pl.BlockSpec((RPB, L), lambda i: (i, 0)),
            pl.BlockSpec((RPB, L), lambda i: (i + NI, 0)),
            pl.BlockSpec((RPB, L), lambda i: (i + 2 * NI, 0)),
            pl.BlockSpec((RPB, 128), lambda i: (0, 0)),
        ],
        out_specs=[
            pl.BlockSpec((1, RPB, 128), lambda i: (i, 0, 0)),
            pl.BlockSpec((1, RPB, 128), lambda i: (i, 0, 0)),
            pl.BlockSpec((1, RPB, 128), lambda i: (i, 0, 0)),
        ],
        out_shape=[
            jax.ShapeDtypeStruct((NI, RPB, 128), jnp.float32),
            jax.ShapeDtypeStruct((NI, RPB, 128), jnp.float32),
            jax.ShapeDtypeStruct((NI, RPB, 128), jnp.float32),
        ],
        scratch_shapes=[pltpu.VMEM((RPB, L), jnp.float32)],
    )(seqT, seqT, seqT, len_bc)


def kernel(sequences, lengths):
    seqT = sequences.transpose(2, 0, 1).reshape(PR, L)
    lenf = lengths.astype(jnp.float32)
    len_bc = jnp.broadcast_to(
        jnp.tile(lenf, RPB // B)[:, None], (RPB, 128))
    ta, tb, tc3 = _mean_tc(seqT, len_bc)
    phys = jnp.concatenate([ta[:, :, 0].reshape(PR // 3),
                            tb[:, :, 0].reshape(PR // 3),
                            tc3[:, :, 0].reshape(PR // 3)])
    return phys.reshape(D, B).T
